# Initial kernel scaffold; baseline (speedup 1.0000x reference)
import jax
import jax.numpy as jnp
from jax.experimental import pallas as pl


def kernel(x, edge_index, edge_attr, W_i, W_h, W_o, b_o):
    raise NotImplementedError("write your pallas kernel here")


# speedup vs baseline: 2.8045x; 2.8045x over previous
import functools

import jax
import jax.numpy as jnp
from jax import lax
from jax.experimental import pallas as pl
from jax.experimental.pallas import tpu as pltpu
from jax.experimental.pallas import tpu_sc as plsc

DIM = 128
_CHG = 640
_IPCG = _CHG // 128
_CHS = 256
_IPCS = _CHS // 128
_NW = 32

_MESH = dict(core_axis_name="c", subcore_axis_name="s")


@functools.lru_cache(maxsize=None)
def _make_gather(V, B):
    nch = B // _CHG
    maxk = (nch + _NW - 1) // _NW

    @functools.partial(
        pl.kernel,
        mesh=plsc.VectorSubcoreMesh(**_MESH),
        out_type=jax.ShapeDtypeStruct((B, DIM), jnp.float32),
        scratch_types=[
            pltpu.VMEM((_IPCG, 128), jnp.int32),
            pltpu.VMEM((_CHG, DIM), jnp.float32),
            pltpu.SemaphoreType.DMA,
        ],
    )
    def gk(table, idx, out, idx_v, rows_v, sem):
        w = lax.axis_index("c") * 16 + lax.axis_index("s")

        def body(k, carry):
            c = w + _NW * k

            @pl.when(c < nch)
            def _():
                pltpu.sync_copy(idx.at[c], idx_v)
                handles = [
                    pltpu.async_copy(
                        table.at[idx_v.at[j]],
                        rows_v.at[pl.ds(j * 128, 128)],
                        sem,
                    )
                    for j in range(_IPCG)
                ]
                for h in handles:
                    h.wait()
                pltpu.sync_copy(rows_v, out.at[pl.ds(c * _CHG, _CHG)])

            return carry

        lax.fori_loop(0, maxk, body, 0)

    return gk


@functools.lru_cache(maxsize=None)
def _make_scatter(V, B):
    nch = B // _CHS
    maxk = (nch + _NW - 1) // _NW
    rpt = (V // 16) // 8 * 8
    rem = V - 16 * rpt

    @functools.partial(
        pl.kernel,
        mesh=plsc.VectorSubcoreMesh(**_MESH),
        out_type=jax.ShapeDtypeStruct((2, V, DIM), jnp.float32),
        scratch_types=[
            pltpu.VMEM((_IPCS, 128), jnp.int32),
            pltpu.VMEM((_CHS, DIM), jnp.float32),
            pltpu.VMEM_SHARED((V, DIM), jnp.float32),
        ],
    )
    def sk(rows_hbm, idx_hbm, zeros_hbm, out, idx_v, rows_v, table):
        cid = lax.axis_index("c")
        sid = lax.axis_index("s")
        w = cid * 16 + sid
        pltpu.sync_copy(
            zeros_hbm.at[pl.ds(sid * rpt, rpt)], table.at[pl.ds(sid * rpt, rpt)]
        )
        if rem:
            @pl.when(sid == 15)
            def _zrem():
                pltpu.sync_copy(
                    zeros_hbm.at[pl.ds(16 * rpt, rem)],
                    table.at[pl.ds(16 * rpt, rem)],
                )
        plsc.subcore_barrier()

        def body(k, carry):
            c = w + _NW * k

            @pl.when(c < nch)
            def _():
                pltpu.sync_copy(idx_hbm.at[c], idx_v)
                pltpu.sync_copy(rows_hbm.at[pl.ds(c * _CHS, _CHS)], rows_v)
                for j in range(_IPCS):
                    pltpu.sync_copy(
                        rows_v.at[pl.ds(j * 128, 128)],
                        table.at[idx_v.at[j]],
                        add=True,
                    )

            return carry

        lax.fori_loop(0, maxk, body, 0)
        plsc.subcore_barrier()
        pltpu.sync_copy(
            table.at[pl.ds(sid * rpt, rpt)], out.at[cid, pl.ds(sid * rpt, rpt)]
        )
        if rem:
            @pl.when(sid == 15)
            def _frem():
                pltpu.sync_copy(
                    table.at[pl.ds(16 * rpt, rem)],
                    out.at[cid, pl.ds(16 * rpt, rem)],
                )

    return sk


_BT = 3200


def _init_body(gx_ref, ea_ref, wx_ref, we_ref, inp_ref, msg_ref):
    acc = jnp.dot(gx_ref[...], wx_ref[...], preferred_element_type=jnp.float32)
    acc = acc + jnp.dot(ea_ref[...], we_ref[...], preferred_element_type=jnp.float32)
    inp_ref[...] = acc
    msg_ref[...] = jnp.maximum(acc, 0.0)


def _tc_init(gx, ea, wx, we):
    e = gx.shape[0]
    return pl.pallas_call(
        _init_body,
        grid=(e // _BT,),
        in_specs=[
            pl.BlockSpec((_BT, DIM), lambda i: (i, 0)),
            pl.BlockSpec((_BT, 16), lambda i: (i, 0)),
            pl.BlockSpec((DIM, DIM), lambda i: (0, 0)),
            pl.BlockSpec((16, DIM), lambda i: (0, 0)),
        ],
        out_specs=[
            pl.BlockSpec((_BT, DIM), lambda i: (i, 0)),
            pl.BlockSpec((_BT, DIM), lambda i: (i, 0)),
        ],
        out_shape=[
            jax.ShapeDtypeStruct((e, DIM), jnp.float32),
            jax.ShapeDtypeStruct((e, DIM), jnp.float32),
        ],
    )(gx, ea, wx, we)


def _depth_body(msg_ref, g_ref, inp_ref, wh_ref, out_ref):
    msg = msg_ref[...]
    fwd = jnp.roll(msg, -1, axis=0)
    bwd = jnp.roll(msg, 1, axis=0)
    row = lax.broadcasted_iota(jnp.int32, msg.shape, 0)
    swapped = jnp.where((row & 1) == 0, fwd, bwd)
    t = g_ref[...] - swapped
    z = inp_ref[...] + jnp.dot(t, wh_ref[...], preferred_element_type=jnp.float32)
    out_ref[...] = jnp.maximum(z, 0.0)


def _tc_depth(msg, g, inp, wh_t):
    e = msg.shape[0]
    return pl.pallas_call(
        _depth_body,
        grid=(e // _BT,),
        in_specs=[
            pl.BlockSpec((_BT, DIM), lambda i: (i, 0)),
            pl.BlockSpec((_BT, DIM), lambda i: (i, 0)),
            pl.BlockSpec((_BT, DIM), lambda i: (i, 0)),
            pl.BlockSpec((DIM, DIM), lambda i: (0, 0)),
        ],
        out_specs=pl.BlockSpec((_BT, DIM), lambda i: (i, 0)),
        out_shape=jax.ShapeDtypeStruct((e, DIM), jnp.float32),
    )(msg, g, inp, wh_t)


def _combine_body(p_ref, out_ref):
    out_ref[...] = p_ref[0] + p_ref[1]


def _tc_combine(part):
    n = part.shape[1]
    bn = 1000
    return pl.pallas_call(
        _combine_body,
        grid=(n // bn,),
        in_specs=[pl.BlockSpec((2, bn, DIM), lambda i: (0, i, 0))],
        out_specs=pl.BlockSpec((bn, DIM), lambda i: (i, 0)),
        out_shape=jax.ShapeDtypeStruct((n, DIM), jnp.float32),
    )(part)


def _final_body(x_ref, p_ref, wox_ref, wos_ref, b_ref, out_ref):
    s = p_ref[0] + p_ref[1]
    z = jnp.dot(x_ref[...], wox_ref[...], preferred_element_type=jnp.float32)
    z = z + jnp.dot(s, wos_ref[...], preferred_element_type=jnp.float32)
    out_ref[...] = jnp.maximum(z + b_ref[...], 0.0)


def _tc_final(x, part, wox, wos, b2):
    n = x.shape[0]
    bn = 1000
    return pl.pallas_call(
        _final_body,
        grid=(n // bn,),
        in_specs=[
            pl.BlockSpec((bn, DIM), lambda i: (i, 0)),
            pl.BlockSpec((2, bn, DIM), lambda i: (0, i, 0)),
            pl.BlockSpec((DIM, DIM), lambda i: (0, 0)),
            pl.BlockSpec((DIM, DIM), lambda i: (0, 0)),
            pl.BlockSpec((1, DIM), lambda i: (0, 0)),
        ],
        out_specs=pl.BlockSpec((bn, DIM), lambda i: (i, 0)),
        out_shape=jax.ShapeDtypeStruct((n, DIM), jnp.float32),
    )(x, part, wox, wos, b2)


def kernel(x, edge_index, edge_attr, W_i, W_h, W_o, b_o):
    n = x.shape[0]
    e = edge_attr.shape[0]
    depth = 6

    src = edge_index[0].astype(jnp.int32)
    dst = edge_index[1].astype(jnp.int32)
    dst_s = dst.reshape(-1, 2)[:, ::-1].reshape(-1)
    src2d = src.reshape(-1, _IPCG, 128)
    dst2d = dst.reshape(-1, _IPCS, 128)
    dst_s2d = dst_s.reshape(-1, _IPCG, 128)
    zeros_tab = jnp.zeros((n, DIM), jnp.float32)

    wx = W_i[:, :DIM].T
    we = W_i[:, DIM:].T
    wh_t = W_h.T
    wox = W_o[:, :DIM].T
    wos = W_o[:, DIM:].T
    b2 = b_o.reshape(1, DIM)

    gather = _make_gather(n, e)
    scatter = _make_scatter(n, e)

    gx = gather(x, src2d)
    inp, msg = _tc_init(gx, edge_attr, wx, we)
    for _ in range(depth - 1):
        part = scatter(msg, dst2d, zeros_tab)
        esum = _tc_combine(part)
        g = gather(esum, dst_s2d)
        msg = _tc_depth(msg, g, inp, wh_t)
    part = scatter(msg, dst2d, zeros_tab)
    return _tc_final(x, part, wox, wos, b2)


# speedup vs baseline: 3.0140x; 1.0747x over previous
import functools

import jax
import jax.numpy as jnp
from jax import lax
from jax.experimental import pallas as pl
from jax.experimental.pallas import tpu as pltpu
from jax.experimental.pallas import tpu_sc as plsc

DIM = 128
_CHG = 256
_IPCG = _CHG // 128
_CHS = 128
_IPCS = _CHS // 128
_NW = 32

_MESH = dict(core_axis_name="c", subcore_axis_name="s")


@functools.lru_cache(maxsize=None)
def _make_gather(V, B):
    nch = B // _CHG
    kmain = nch // _NW
    nextra = nch - kmain * _NW
    rbytes = _CHG * DIM * 4

    @functools.partial(
        pl.kernel,
        mesh=plsc.VectorSubcoreMesh(**_MESH),
        out_type=jax.ShapeDtypeStruct((B, DIM), jnp.float32),
        scratch_types=[
            pltpu.VMEM((kmain, _IPCG, 128), jnp.int32),
            pltpu.VMEM((_IPCG, 128), jnp.int32),
            pltpu.VMEM((2, _CHG, DIM), jnp.float32),
            pltpu.SemaphoreType.DMA,
            pltpu.SemaphoreType.DMA,
            pltpu.SemaphoreType.DMA,
        ],
    )
    def gk(table, idx, out, idx_all, idx_x, rows_v, sem_g, sem_o0, sem_o1):
        w = lax.axis_index("c") * 16 + lax.axis_index("s")
        base = w * kmain
        pltpu.sync_copy(idx.at[pl.ds(base, kmain)], idx_all)
        if nextra:
            @pl.when(w < nextra)
            def _():
                pltpu.sync_copy(idx.at[_NW * kmain + w], idx_x)

        def chunk(k, buf, sem):
            hs = [
                pltpu.async_copy(
                    table.at[idx_all.at[k, j]],
                    rows_v.at[buf, pl.ds(j * 128, 128)],
                    sem_g,
                )
                for j in range(_IPCG)
            ]
            for h in hs:
                h.wait()
            pltpu.async_copy(
                rows_v.at[buf], out.at[pl.ds((base + k) * _CHG, _CHG)], sem
            )

        def drain(sem, buf):
            pltpu.make_async_copy(
                out.at[pl.ds(0, _CHG)], rows_v.at[buf], sem
            ).wait()

        def body(i, carry):
            @pl.when(i > 0)
            def _():
                drain(sem_o0, 0)
            chunk(2 * i, 0, sem_o0)

            @pl.when(i > 0)
            def _():
                drain(sem_o1, 1)
            chunk(2 * i + 1, 1, sem_o1)
            return carry

        npair = kmain // 2
        lax.fori_loop(0, npair, body, 0)
        if kmain % 2:
            drain(sem_o0, 0)
            chunk(kmain - 1, 0, sem_o0)
        if nextra:
            @pl.when(w < nextra)
            def _():
                drain(sem_o1, 1)
                hs = [
                    pltpu.async_copy(
                        table.at[idx_x.at[j]],
                        rows_v.at[1, pl.ds(j * 128, 128)],
                        sem_g,
                    )
                    for j in range(_IPCG)
                ]
                for h in hs:
                    h.wait()
                pltpu.async_copy(
                    rows_v.at[1],
                    out.at[pl.ds((_NW * kmain + w) * _CHG, _CHG)],
                    sem_o1,
                )
        drain(sem_o0, 0)
        drain(sem_o1, 1)

    return gk


@functools.lru_cache(maxsize=None)
def _make_scatter(V, B):
    nch = B // _CHS
    kmain = nch // _NW
    nextra = nch - kmain * _NW
    rpt = (V // 16) // 8 * 8
    rem = V - 16 * rpt

    @functools.partial(
        pl.kernel,
        mesh=plsc.VectorSubcoreMesh(**_MESH),
        out_type=jax.ShapeDtypeStruct((2, V, DIM), jnp.float32),
        scratch_types=[
            pltpu.VMEM((kmain, _IPCS, 128), jnp.int32),
            pltpu.VMEM((_IPCS, 128), jnp.int32),
            pltpu.VMEM((2, _CHS, DIM), jnp.float32),
            pltpu.VMEM_SHARED((V, DIM), jnp.float32),
            pltpu.SemaphoreType.DMA,
            pltpu.SemaphoreType.DMA,
        ],
    )
    def sk(rows_hbm, idx_hbm, zeros_hbm, out, idx_all, idx_x, rows_v, table,
           sem_l0, sem_l1):
        cid = lax.axis_index("c")
        sid = lax.axis_index("s")
        w = cid * 16 + sid
        base = w * kmain
        sems = (sem_l0, sem_l1)

        def fire(k, buf):
            pltpu.async_copy(
                rows_hbm.at[pl.ds((base + k) * _CHS, _CHS)],
                rows_v.at[buf],
                sems[buf],
            )

        def drain(buf):
            pltpu.make_async_copy(
                rows_hbm.at[pl.ds(0, _CHS)], rows_v.at[buf], sems[buf]
            ).wait()

        pltpu.sync_copy(idx_hbm.at[pl.ds(base, kmain)], idx_all)
        if nextra:
            @pl.when(w < nextra)
            def _():
                pltpu.sync_copy(idx_hbm.at[_NW * kmain + w], idx_x)
        fire(0, 0)
        pltpu.sync_copy(
            zeros_hbm.at[pl.ds(sid * rpt, rpt)], table.at[pl.ds(sid * rpt, rpt)]
        )
        if rem:
            @pl.when(sid == 15)
            def _():
                pltpu.sync_copy(
                    zeros_hbm.at[pl.ds(16 * rpt, rem)],
                    table.at[pl.ds(16 * rpt, rem)],
                )
        plsc.subcore_barrier()

        def scat(k, buf):
            for j in range(_IPCS):
                pltpu.sync_copy(
                    rows_v.at[buf, pl.ds(j * 128, 128)],
                    table.at[idx_all.at[k, j]],
                    add=True,
                )

        def body(i, carry):
            k0 = 2 * i
            drain(0)
            fire(k0 + 1, 1)
            scat(k0, 0)
            drain(1)

            @pl.when(k0 + 2 < kmain)
            def _():
                fire(k0 + 2, 0)
            scat(k0 + 1, 1)
            return carry

        lax.fori_loop(0, kmain // 2, body, 0)
        if nextra:
            @pl.when(w < nextra)
            def _():
                c = _NW * kmain + w
                pltpu.sync_copy(rows_hbm.at[pl.ds(c * _CHS, _CHS)], rows_v.at[0])
                for j in range(_IPCS):
                    pltpu.sync_copy(
                        rows_v.at[0, pl.ds(j * 128, 128)],
                        table.at[idx_x.at[j]],
                        add=True,
                    )
        plsc.subcore_barrier()
        pltpu.sync_copy(
            table.at[pl.ds(sid * rpt, rpt)], out.at[cid, pl.ds(sid * rpt, rpt)]
        )
        if rem:
            @pl.when(sid == 15)
            def _():
                pltpu.sync_copy(
                    table.at[pl.ds(16 * rpt, rem)],
                    out.at[cid, pl.ds(16 * rpt, rem)],
                )

    return sk


_BT = 3200


def _init_body(gx_ref, ea_ref, wx_ref, we_ref, inp_ref, msg_ref):
    acc = jnp.dot(gx_ref[...], wx_ref[...], preferred_element_type=jnp.float32)
    acc = acc + jnp.dot(ea_ref[...], we_ref[...], preferred_element_type=jnp.float32)
    inp_ref[...] = acc
    msg_ref[...] = jnp.maximum(acc, 0.0)


def _tc_init(gx, ea, wx, we):
    e = gx.shape[0]
    return pl.pallas_call(
        _init_body,
        grid=(e // _BT,),
        in_specs=[
            pl.BlockSpec((_BT, DIM), lambda i: (i, 0)),
            pl.BlockSpec((_BT, 16), lambda i: (i, 0)),
            pl.BlockSpec((DIM, DIM), lambda i: (0, 0)),
            pl.BlockSpec((16, DIM), lambda i: (0, 0)),
        ],
        out_specs=[
            pl.BlockSpec((_BT, DIM), lambda i: (i, 0)),
            pl.BlockSpec((_BT, DIM), lambda i: (i, 0)),
        ],
        out_shape=[
            jax.ShapeDtypeStruct((e, DIM), jnp.float32),
            jax.ShapeDtypeStruct((e, DIM), jnp.float32),
        ],
    )(gx, ea, wx, we)


def _depth_body(msg_ref, g_ref, inp_ref, wh_ref, out_ref):
    msg = msg_ref[...]
    fwd = jnp.roll(msg, -1, axis=0)
    bwd = jnp.roll(msg, 1, axis=0)
    row = lax.broadcasted_iota(jnp.int32, msg.shape, 0)
    swapped = jnp.where((row & 1) == 0, fwd, bwd)
    t = g_ref[...] - swapped
    z = inp_ref[...] + jnp.dot(t, wh_ref[...], preferred_element_type=jnp.float32)
    out_ref[...] = jnp.maximum(z, 0.0)


def _tc_depth(msg, g, inp, wh_t):
    e = msg.shape[0]
    return pl.pallas_call(
        _depth_body,
        grid=(e // _BT,),
        in_specs=[
            pl.BlockSpec((_BT, DIM), lambda i: (i, 0)),
            pl.BlockSpec((_BT, DIM), lambda i: (i, 0)),
            pl.BlockSpec((_BT, DIM), lambda i: (i, 0)),
            pl.BlockSpec((DIM, DIM), lambda i: (0, 0)),
        ],
        out_specs=pl.BlockSpec((_BT, DIM), lambda i: (i, 0)),
        out_shape=jax.ShapeDtypeStruct((e, DIM), jnp.float32),
    )(msg, g, inp, wh_t)


def _combine_body(p_ref, out_ref):
    out_ref[...] = p_ref[0] + p_ref[1]


def _tc_combine(part):
    n = part.shape[1]
    bn = 1000
    return pl.pallas_call(
        _combine_body,
        grid=(n // bn,),
        in_specs=[pl.BlockSpec((2, bn, DIM), lambda i: (0, i, 0))],
        out_specs=pl.BlockSpec((bn, DIM), lambda i: (i, 0)),
        out_shape=jax.ShapeDtypeStruct((n, DIM), jnp.float32),
    )(part)


def _final_body(x_ref, p_ref, wox_ref, wos_ref, b_ref, out_ref):
    s = p_ref[0] + p_ref[1]
    z = jnp.dot(x_ref[...], wox_ref[...], preferred_element_type=jnp.float32)
    z = z + jnp.dot(s, wos_ref[...], preferred_element_type=jnp.float32)
    out_ref[...] = jnp.maximum(z + b_ref[...], 0.0)


def _tc_final(x, part, wox, wos, b2):
    n = x.shape[0]
    bn = 1000
    return pl.pallas_call(
        _final_body,
        grid=(n // bn,),
        in_specs=[
            pl.BlockSpec((bn, DIM), lambda i: (i, 0)),
            pl.BlockSpec((2, bn, DIM), lambda i: (0, i, 0)),
            pl.BlockSpec((DIM, DIM), lambda i: (0, 0)),
            pl.BlockSpec((DIM, DIM), lambda i: (0, 0)),
            pl.BlockSpec((1, DIM), lambda i: (0, 0)),
        ],
        out_specs=pl.BlockSpec((bn, DIM), lambda i: (i, 0)),
        out_shape=jax.ShapeDtypeStruct((n, DIM), jnp.float32),
    )(x, part, wox, wos, b2)


def kernel(x, edge_index, edge_attr, W_i, W_h, W_o, b_o):
    n = x.shape[0]
    e = edge_attr.shape[0]
    depth = 6

    src = edge_index[0].astype(jnp.int32)
    dst = edge_index[1].astype(jnp.int32)
    dst_s = dst.reshape(-1, 2)[:, ::-1].reshape(-1)
    src2d = src.reshape(-1, _IPCG, 128)
    dst2d = dst.reshape(-1, _IPCS, 128)
    dst_s2d = dst_s.reshape(-1, _IPCG, 128)
    zeros_tab = jnp.zeros((n, DIM), jnp.float32)

    wx = W_i[:, :DIM].T
    we = W_i[:, DIM:].T
    wh_t = W_h.T
    wox = W_o[:, :DIM].T
    wos = W_o[:, DIM:].T
    b2 = b_o.reshape(1, DIM)

    gather = _make_gather(n, e)
    scatter = _make_scatter(n, e)

    gx = gather(x, src2d)
    inp, msg = _tc_init(gx, edge_attr, wx, we)
    for _ in range(depth - 1):
        part = scatter(msg, dst2d, zeros_tab)
        esum = _tc_combine(part)
        g = gather(esum, dst_s2d)
        msg = _tc_depth(msg, g, inp, wh_t)
    part = scatter(msg, dst2d, zeros_tab)
    return _tc_final(x, part, wox, wos, b2)


# speedup vs baseline: 3.7723x; 1.2516x over previous
import functools

import jax
import jax.numpy as jnp
from jax import lax
from jax.experimental import pallas as pl
from jax.experimental.pallas import tpu as pltpu
from jax.experimental.pallas import tpu_sc as plsc

DIM = 128
_CHG = 256
_IPCG = _CHG // 128
_CHS = 128
_IPCS = _CHS // 128
_NW = 32

_MESH = dict(core_axis_name="c", subcore_axis_name="s")


@functools.lru_cache(maxsize=None)
def _make_gather(V, B):
    rreal = B // 128
    rpw = (rreal + _NW - 1) // _NW
    rpw = (rpw + 7) // 8 * 8
    nk = rpw // _IPCG

    @functools.partial(
        pl.kernel,
        mesh=plsc.VectorSubcoreMesh(**_MESH),
        out_type=jax.ShapeDtypeStruct((B, DIM), jnp.float32),
        scratch_types=[
            pltpu.VMEM((rpw, 128), jnp.int32),
            pltpu.VMEM((2, _CHG, DIM), jnp.float32),
            pltpu.SemaphoreType.DMA,
            pltpu.SemaphoreType.DMA,
            pltpu.SemaphoreType.DMA,
        ],
    )
    def gk(table, idx, out, idx_all, rows_v, sem_g, sem_o0, sem_o1):
        w = lax.axis_index("c") * 16 + lax.axis_index("s")
        row0 = w * rpw
        pltpu.sync_copy(idx.at[pl.ds(row0, rpw)], idx_all)
        nvalid = jnp.minimum(nk, (rreal - row0) // _IPCG)

        def chunk(k, buf, sem):
            hs = [
                pltpu.async_copy(
                    table.at[idx_all.at[_IPCG * k + j]],
                    rows_v.at[buf, pl.ds(j * 128, 128)],
                    sem_g,
                )
                for j in range(_IPCG)
            ]
            for h in hs:
                h.wait()
            pltpu.async_copy(
                rows_v.at[buf], out.at[pl.ds((row0 + _IPCG * k) * 128, _CHG)], sem
            )

        def drain(sem, buf):
            pltpu.make_async_copy(
                out.at[pl.ds(0, _CHG)], rows_v.at[buf], sem
            ).wait()

        def body(i, carry):
            k0 = 2 * i

            @pl.when(k0 < nvalid)
            def _():
                @pl.when(i > 0)
                def _():
                    drain(sem_o0, 0)
                chunk(k0, 0, sem_o0)

                @pl.when(i > 0)
                def _():
                    drain(sem_o1, 1)
                chunk(k0 + 1, 1, sem_o1)

            return carry

        lax.fori_loop(0, nk // 2, body, 0)
        drain(sem_o0, 0)
        drain(sem_o1, 1)

    return gk


@functools.lru_cache(maxsize=None)
def _make_scatter(V, B):
    rreal = B // 128
    rpw = (rreal + _NW - 1) // _NW
    rpw = (rpw + 7) // 8 * 8
    nk = rpw // _IPCS
    rpt = (V // 16) // 8 * 8
    rem = V - 16 * rpt

    @functools.partial(
        pl.kernel,
        mesh=plsc.VectorSubcoreMesh(**_MESH),
        out_type=jax.ShapeDtypeStruct((2, V, DIM), jnp.float32),
        scratch_types=[
            pltpu.VMEM((rpw, 128), jnp.int32),
            pltpu.VMEM((2, _CHS, DIM), jnp.float32),
            pltpu.VMEM_SHARED((V, DIM), jnp.float32),
            pltpu.SemaphoreType.DMA,
            pltpu.SemaphoreType.DMA,
        ],
    )
    def sk(rows_hbm, idx_hbm, zeros_hbm, out, idx_all, rows_v, table,
           sem_l0, sem_l1):
        cid = lax.axis_index("c")
        sid = lax.axis_index("s")
        w = cid * 16 + sid
        row0 = w * rpw
        nvalid = jnp.minimum(nk, (rreal - row0) // _IPCS)
        sems = (sem_l0, sem_l1)

        def fire(k, buf):
            pltpu.async_copy(
                rows_hbm.at[pl.ds((row0 + _IPCS * k) * 128, _CHS)],
                rows_v.at[buf],
                sems[buf],
            )

        def drain(buf):
            pltpu.make_async_copy(
                rows_hbm.at[pl.ds(0, _CHS)], rows_v.at[buf], sems[buf]
            ).wait()

        pltpu.sync_copy(idx_hbm.at[pl.ds(row0, rpw)], idx_all)
        fire(0, 0)
        pltpu.sync_copy(
            zeros_hbm.at[pl.ds(sid * rpt, rpt)], table.at[pl.ds(sid * rpt, rpt)]
        )
        if rem:
            @pl.when(sid == 15)
            def _():
                pltpu.sync_copy(
                    zeros_hbm.at[pl.ds(16 * rpt, rem)],
                    table.at[pl.ds(16 * rpt, rem)],
                )
        plsc.subcore_barrier()

        def scat(k, buf):
            for j in range(_IPCS):
                pltpu.sync_copy(
                    rows_v.at[buf, pl.ds(j * 128, 128)],
                    table.at[idx_all.at[_IPCS * k + j]],
                    add=True,
                )

        def body(i, carry):
            k0 = 2 * i

            @pl.when(k0 < nvalid)
            def _():
                drain(0)

                @pl.when(k0 + 1 < nvalid)
                def _():
                    fire(k0 + 1, 1)
                scat(k0, 0)

                @pl.when(k0 + 1 < nvalid)
                def _():
                    drain(1)

                    @pl.when(k0 + 2 < nvalid)
                    def _():
                        fire(k0 + 2, 0)
                    scat(k0 + 1, 1)

            return carry

        lax.fori_loop(0, nk // 2, body, 0)
        plsc.subcore_barrier()
        pltpu.sync_copy(
            table.at[pl.ds(sid * rpt, rpt)], out.at[cid, pl.ds(sid * rpt, rpt)]
        )
        if rem:
            @pl.when(sid == 15)
            def _():
                pltpu.sync_copy(
                    table.at[pl.ds(16 * rpt, rem)],
                    out.at[cid, pl.ds(16 * rpt, rem)],
                )

    return sk


_BT = 3200


def _init_body(gx_ref, ea_ref, wx_ref, we_ref, inp_ref, msg_ref):
    acc = jnp.dot(gx_ref[...], wx_ref[...], preferred_element_type=jnp.float32)
    acc = acc + jnp.dot(ea_ref[...], we_ref[...], preferred_element_type=jnp.float32)
    inp_ref[...] = acc
    msg_ref[...] = jnp.maximum(acc, 0.0)


def _tc_init(gx, ea, wx, we):
    e = gx.shape[0]
    return pl.pallas_call(
        _init_body,
        grid=(e // _BT,),
        in_specs=[
            pl.BlockSpec((_BT, DIM), lambda i: (i, 0)),
            pl.BlockSpec((_BT, 16), lambda i: (i, 0)),
            pl.BlockSpec((DIM, DIM), lambda i: (0, 0)),
            pl.BlockSpec((16, DIM), lambda i: (0, 0)),
        ],
        out_specs=[
            pl.BlockSpec((_BT, DIM), lambda i: (i, 0)),
            pl.BlockSpec((_BT, DIM), lambda i: (i, 0)),
        ],
        out_shape=[
            jax.ShapeDtypeStruct((e, DIM), jnp.float32),
            jax.ShapeDtypeStruct((e, DIM), jnp.float32),
        ],
    )(gx, ea, wx, we)


def _depth_body(msg_ref, g_ref, inp_ref, wh_ref, out_ref):
    msg = msg_ref[...]
    fwd = jnp.roll(msg, -1, axis=0)
    bwd = jnp.roll(msg, 1, axis=0)
    row = lax.broadcasted_iota(jnp.int32, msg.shape, 0)
    swapped = jnp.where((row & 1) == 0, fwd, bwd)
    t = g_ref[...] - swapped
    z = inp_ref[...] + jnp.dot(t, wh_ref[...], preferred_element_type=jnp.float32)
    out_ref[...] = jnp.maximum(z, 0.0)


def _tc_depth(msg, g, inp, wh_t):
    e = msg.shape[0]
    return pl.pallas_call(
        _depth_body,
        grid=(e // _BT,),
        in_specs=[
            pl.BlockSpec((_BT, DIM), lambda i: (i, 0)),
            pl.BlockSpec((_BT, DIM), lambda i: (i, 0)),
            pl.BlockSpec((_BT, DIM), lambda i: (i, 0)),
            pl.BlockSpec((DIM, DIM), lambda i: (0, 0)),
        ],
        out_specs=pl.BlockSpec((_BT, DIM), lambda i: (i, 0)),
        out_shape=jax.ShapeDtypeStruct((e, DIM), jnp.float32),
    )(msg, g, inp, wh_t)


def _combine_body(p_ref, out_ref):
    out_ref[...] = p_ref[0] + p_ref[1]


def _tc_combine(part):
    n = part.shape[1]
    bn = 1000
    return pl.pallas_call(
        _combine_body,
        grid=(n // bn,),
        in_specs=[pl.BlockSpec((2, bn, DIM), lambda i: (0, i, 0))],
        out_specs=pl.BlockSpec((bn, DIM), lambda i: (i, 0)),
        out_shape=jax.ShapeDtypeStruct((n, DIM), jnp.float32),
    )(part)


def _final_body(x_ref, p_ref, wox_ref, wos_ref, b_ref, out_ref):
    s = p_ref[0] + p_ref[1]
    z = jnp.dot(x_ref[...], wox_ref[...], preferred_element_type=jnp.float32)
    z = z + jnp.dot(s, wos_ref[...], preferred_element_type=jnp.float32)
    out_ref[...] = jnp.maximum(z + b_ref[...], 0.0)


def _tc_final(x, part, wox, wos, b2):
    n = x.shape[0]
    bn = 1000
    return pl.pallas_call(
        _final_body,
        grid=(n // bn,),
        in_specs=[
            pl.BlockSpec((bn, DIM), lambda i: (i, 0)),
            pl.BlockSpec((2, bn, DIM), lambda i: (0, i, 0)),
            pl.BlockSpec((DIM, DIM), lambda i: (0, 0)),
            pl.BlockSpec((DIM, DIM), lambda i: (0, 0)),
            pl.BlockSpec((1, DIM), lambda i: (0, 0)),
        ],
        out_specs=pl.BlockSpec((bn, DIM), lambda i: (i, 0)),
        out_shape=jax.ShapeDtypeStruct((n, DIM), jnp.float32),
    )(x, part, wox, wos, b2)


def _prep_body(ei_ref, src_ref, dst_ref, dsts_ref):
    s = ei_ref[0]
    d = ei_ref[1]
    fwd = jnp.roll(d, -1, axis=1)
    bwd = jnp.roll(d, 1, axis=1)
    lane = lax.broadcasted_iota(jnp.int32, d.shape, 1)
    ds_ = jnp.where((lane & 1) == 0, fwd, bwd)
    pad = jnp.zeros((src_ref.shape[0] - s.shape[0], 128), jnp.int32)
    src_ref[...] = jnp.concatenate([s, pad], axis=0)
    dst_ref[...] = jnp.concatenate([d, pad], axis=0)
    dsts_ref[...] = jnp.concatenate([ds_, pad], axis=0)


def _tc_prep(ei3, rpad):
    r = ei3.shape[1]
    return pl.pallas_call(
        _prep_body,
        grid=(1,),
        in_specs=[pl.BlockSpec((2, r, 128), lambda i: (0, 0, 0))],
        out_specs=[pl.BlockSpec((rpad, 128), lambda i: (0, 0))] * 3,
        out_shape=[jax.ShapeDtypeStruct((rpad, 128), jnp.int32)] * 3,
    )(ei3)


def kernel(x, edge_index, edge_attr, W_i, W_h, W_o, b_o):
    n = x.shape[0]
    e = edge_attr.shape[0]
    depth = 6

    r = e // 128
    rpw = ((r + _NW - 1) // _NW + 7) // 8 * 8
    rpad = rpw * _NW
    ei3 = edge_index.astype(jnp.int32).reshape(2, r, 128)
    src_i, dst_i, dsts_i = _tc_prep(ei3, rpad)
    zeros_tab = jnp.zeros((n, DIM), jnp.float32)

    wx = W_i[:, :DIM].T
    we = W_i[:, DIM:].T
    wh_t = W_h.T
    wox = W_o[:, :DIM].T
    wos = W_o[:, DIM:].T
    b2 = b_o.reshape(1, DIM)

    gather = _make_gather(n, e)
    scatter = _make_scatter(n, e)

    gx = gather(x, src_i)
    inp, msg = _tc_init(gx, edge_attr, wx, we)
    for _ in range(depth - 1):
        part = scatter(msg, dst_i, zeros_tab)
        esum = _tc_combine(part)
        g = gather(esum, dsts_i)
        msg = _tc_depth(msg, g, inp, wh_t)
    part = scatter(msg, dst_i, zeros_tab)
    return _tc_final(x, part, wox, wos, b2)


# speedup vs baseline: 3.9541x; 1.0482x over previous
import functools

import jax
import jax.numpy as jnp
from jax import lax
from jax.experimental import pallas as pl
from jax.experimental.pallas import tpu as pltpu
from jax.experimental.pallas import tpu_sc as plsc

DIM = 128
_CHG = 256
_IPCG = _CHG // 128
_CHS = 128
_IPCS = _CHS // 128
_NW = 32

_MESH = dict(core_axis_name="c", subcore_axis_name="s")


@functools.lru_cache(maxsize=None)
def _make_gather(V, B, off_rows):
    rreal = B // 128
    rpw = (rreal + _NW - 1) // _NW
    rpw = (rpw + 7) // 8 * 8
    nk = rpw // _IPCG

    @functools.partial(
        pl.kernel,
        mesh=plsc.VectorSubcoreMesh(**_MESH),
        out_type=jax.ShapeDtypeStruct((B, DIM), jnp.float32),
        scratch_types=[
            pltpu.VMEM((rpw, 128), jnp.int32),
            pltpu.VMEM((2, _CHG, DIM), jnp.float32),
            pltpu.SemaphoreType.DMA,
            pltpu.SemaphoreType.DMA,
            pltpu.SemaphoreType.DMA,
        ],
    )
    def gk(table, idx, out, idx_all, rows_v, sem_g, sem_o0, sem_o1):
        w = lax.axis_index("c") * 16 + lax.axis_index("s")
        lrow0 = w * rpw
        pltpu.sync_copy(idx.at[pl.ds(off_rows + lrow0, rpw)], idx_all)
        nvalid = jnp.minimum(nk, (rreal - lrow0) // _IPCG)

        def chunk(k, buf, sem):
            hs = [
                pltpu.async_copy(
                    table.at[idx_all.at[_IPCG * k + j]],
                    rows_v.at[buf, pl.ds(j * 128, 128)],
                    sem_g,
                )
                for j in range(_IPCG)
            ]
            for h in hs:
                h.wait()
            pltpu.async_copy(
                rows_v.at[buf], out.at[pl.ds((lrow0 + _IPCG * k) * 128, _CHG)], sem
            )

        def drain(sem, buf):
            pltpu.make_async_copy(
                out.at[pl.ds(0, _CHG)], rows_v.at[buf], sem
            ).wait()

        def body(i, carry):
            k0 = 2 * i

            @pl.when(k0 < nvalid)
            def _():
                @pl.when(i > 0)
                def _():
                    drain(sem_o0, 0)
                chunk(k0, 0, sem_o0)

                @pl.when(k0 + 1 < nvalid)
                def _():
                    @pl.when(i > 0)
                    def _():
                        drain(sem_o1, 1)
                    chunk(k0 + 1, 1, sem_o1)

            return carry

        lax.fori_loop(0, (nk + 1) // 2, body, 0)
        drain(sem_o0, 0)
        drain(sem_o1, 1)

    return gk


@functools.lru_cache(maxsize=None)
def _make_scatter(V, B, off_rows):
    rreal = B // 128
    rpw = (rreal + _NW - 1) // _NW
    rpw = (rpw + 7) // 8 * 8
    nk = rpw // _IPCS
    rpt = (V // 16) // 8 * 8
    rem = V - 16 * rpt

    @functools.partial(
        pl.kernel,
        mesh=plsc.VectorSubcoreMesh(**_MESH),
        out_type=jax.ShapeDtypeStruct((2, V, DIM), jnp.float32),
        scratch_types=[
            pltpu.VMEM((rpw, 128), jnp.int32),
            pltpu.VMEM((2, _CHS, DIM), jnp.float32),
            pltpu.VMEM_SHARED((V, DIM), jnp.float32),
            pltpu.SemaphoreType.DMA,
            pltpu.SemaphoreType.DMA,
        ],
    )
    def sk(rows_hbm, idx_hbm, zeros_hbm, out, idx_all, rows_v, table,
           sem_l0, sem_l1):
        cid = lax.axis_index("c")
        sid = lax.axis_index("s")
        w = cid * 16 + sid
        lrow0 = w * rpw
        nvalid = jnp.minimum(nk, (rreal - lrow0) // _IPCS)
        sems = (sem_l0, sem_l1)

        def fire(k, buf):
            pltpu.async_copy(
                rows_hbm.at[pl.ds((lrow0 + _IPCS * k) * 128, _CHS)],
                rows_v.at[buf],
                sems[buf],
            )

        def drain(buf):
            pltpu.make_async_copy(
                rows_hbm.at[pl.ds(0, _CHS)], rows_v.at[buf], sems[buf]
            ).wait()

        pltpu.sync_copy(idx_hbm.at[pl.ds(off_rows + lrow0, rpw)], idx_all)
        fire(0, 0)
        pltpu.sync_copy(
            zeros_hbm.at[pl.ds(sid * rpt, rpt)], table.at[pl.ds(sid * rpt, rpt)]
        )
        if rem:
            @pl.when(sid == 15)
            def _():
                pltpu.sync_copy(
                    zeros_hbm.at[pl.ds(16 * rpt, rem)],
                    table.at[pl.ds(16 * rpt, rem)],
                )
        plsc.subcore_barrier()

        def scat(k, buf):
            for j in range(_IPCS):
                pltpu.sync_copy(
                    rows_v.at[buf, pl.ds(j * 128, 128)],
                    table.at[idx_all.at[_IPCS * k + j]],
                    add=True,
                )

        def body(i, carry):
            k0 = 2 * i

            @pl.when(k0 < nvalid)
            def _():
                drain(0)

                @pl.when(k0 + 1 < nvalid)
                def _():
                    fire(k0 + 1, 1)
                scat(k0, 0)

                @pl.when(k0 + 1 < nvalid)
                def _():
                    drain(1)

                    @pl.when(k0 + 2 < nvalid)
                    def _():
                        fire(k0 + 2, 0)
                    scat(k0 + 1, 1)

            return carry

        lax.fori_loop(0, nk // 2, body, 0)
        plsc.subcore_barrier()
        pltpu.sync_copy(
            table.at[pl.ds(sid * rpt, rpt)], out.at[cid, pl.ds(sid * rpt, rpt)]
        )
        if rem:
            @pl.when(sid == 15)
            def _():
                pltpu.sync_copy(
                    table.at[pl.ds(16 * rpt, rem)],
                    out.at[cid, pl.ds(16 * rpt, rem)],
                )

    return sk


_BT = 3200


def _init_body(gx_ref, ea_ref, wx_ref, we_ref, inp_ref, msg_ref):
    acc = jnp.dot(gx_ref[...], wx_ref[...], preferred_element_type=jnp.float32)
    acc = acc + jnp.dot(ea_ref[...], we_ref[...], preferred_element_type=jnp.float32)
    inp_ref[...] = acc
    msg_ref[...] = jnp.maximum(acc, 0.0)


def _tc_init(gx, ea, wx, we, h):
    e = gx.shape[0]
    nb = e // _BT
    return pl.pallas_call(
        _init_body,
        grid=(nb,),
        in_specs=[
            pl.BlockSpec((_BT, DIM), lambda i: (i, 0)),
            pl.BlockSpec((_BT, 16), lambda i, h=h, nb=nb: (i + h * nb, 0)),
            pl.BlockSpec((DIM, DIM), lambda i: (0, 0)),
            pl.BlockSpec((16, DIM), lambda i: (0, 0)),
        ],
        out_specs=[
            pl.BlockSpec((_BT, DIM), lambda i: (i, 0)),
            pl.BlockSpec((_BT, DIM), lambda i: (i, 0)),
        ],
        out_shape=[
            jax.ShapeDtypeStruct((e, DIM), jnp.float32),
            jax.ShapeDtypeStruct((e, DIM), jnp.float32),
        ],
    )(gx, ea, wx, we)


def _depth_body(msg_ref, g_ref, inp_ref, wh_ref, out_ref):
    msg = msg_ref[...]
    fwd = jnp.roll(msg, -1, axis=0)
    bwd = jnp.roll(msg, 1, axis=0)
    row = lax.broadcasted_iota(jnp.int32, msg.shape, 0)
    swapped = jnp.where((row & 1) == 0, fwd, bwd)
    t = g_ref[...] - swapped
    z = inp_ref[...] + jnp.dot(t, wh_ref[...], preferred_element_type=jnp.float32)
    out_ref[...] = jnp.maximum(z, 0.0)


def _tc_depth(msg, g, inp, wh_t):
    e = msg.shape[0]
    return pl.pallas_call(
        _depth_body,
        grid=(e // _BT,),
        in_specs=[
            pl.BlockSpec((_BT, DIM), lambda i: (i, 0)),
            pl.BlockSpec((_BT, DIM), lambda i: (i, 0)),
            pl.BlockSpec((_BT, DIM), lambda i: (i, 0)),
            pl.BlockSpec((DIM, DIM), lambda i: (0, 0)),
        ],
        out_specs=pl.BlockSpec((_BT, DIM), lambda i: (i, 0)),
        out_shape=jax.ShapeDtypeStruct((e, DIM), jnp.float32),
    )(msg, g, inp, wh_t)


def _combine_body(pa_ref, pb_ref, out_ref):
    out_ref[...] = (pa_ref[0] + pa_ref[1]) + (pb_ref[0] + pb_ref[1])


def _tc_combine(pa, pb):
    n = pa.shape[1]
    bn = 1000
    return pl.pallas_call(
        _combine_body,
        grid=(n // bn,),
        in_specs=[pl.BlockSpec((2, bn, DIM), lambda i: (0, i, 0))] * 2,
        out_specs=pl.BlockSpec((bn, DIM), lambda i: (i, 0)),
        out_shape=jax.ShapeDtypeStruct((n, DIM), jnp.float32),
    )(pa, pb)


def _final_body(x_ref, pa_ref, pb_ref, wox_ref, wos_ref, b_ref, out_ref):
    s = (pa_ref[0] + pa_ref[1]) + (pb_ref[0] + pb_ref[1])
    z = jnp.dot(x_ref[...], wox_ref[...], preferred_element_type=jnp.float32)
    z = z + jnp.dot(s, wos_ref[...], preferred_element_type=jnp.float32)
    out_ref[...] = jnp.maximum(z + b_ref[...], 0.0)


def _tc_final(x, pa, pb, wox, wos, b2):
    n = x.shape[0]
    bn = 1000
    return pl.pallas_call(
        _final_body,
        grid=(n // bn,),
        in_specs=[
            pl.BlockSpec((bn, DIM), lambda i: (i, 0)),
            pl.BlockSpec((2, bn, DIM), lambda i: (0, i, 0)),
            pl.BlockSpec((2, bn, DIM), lambda i: (0, i, 0)),
            pl.BlockSpec((DIM, DIM), lambda i: (0, 0)),
            pl.BlockSpec((DIM, DIM), lambda i: (0, 0)),
            pl.BlockSpec((1, DIM), lambda i: (0, 0)),
        ],
        out_specs=pl.BlockSpec((bn, DIM), lambda i: (i, 0)),
        out_shape=jax.ShapeDtypeStruct((n, DIM), jnp.float32),
    )(x, pa, pb, wox, wos, b2)


def _prep_body(ei_ref, src_ref, dst_ref, dsts_ref):
    s = ei_ref[0]
    d = ei_ref[1]
    fwd = jnp.roll(d, -1, axis=1)
    bwd = jnp.roll(d, 1, axis=1)
    lane = lax.broadcasted_iota(jnp.int32, d.shape, 1)
    ds_ = jnp.where((lane & 1) == 0, fwd, bwd)
    r = s.shape[0]
    rh = r // 2
    pad = jnp.zeros((src_ref.shape[0] // 2 - rh, 128), jnp.int32)

    def halved(v):
        return jnp.concatenate([v[:rh], pad, v[rh:], pad], axis=0)

    src_ref[...] = halved(s)
    dst_ref[...] = halved(d)
    dsts_ref[...] = halved(ds_)


def _tc_prep(ei3, rpad2):
    r = ei3.shape[1]
    return pl.pallas_call(
        _prep_body,
        grid=(1,),
        in_specs=[pl.BlockSpec((2, r, 128), lambda i: (0, 0, 0))],
        out_specs=[pl.BlockSpec((rpad2, 128), lambda i: (0, 0))] * 3,
        out_shape=[jax.ShapeDtypeStruct((rpad2, 128), jnp.int32)] * 3,
    )(ei3)


def kernel(x, edge_index, edge_attr, W_i, W_h, W_o, b_o):
    n = x.shape[0]
    e = edge_attr.shape[0]
    eh = e // 2
    depth = 6

    r = e // 128
    rh = r // 2
    rpw = ((rh + _NW - 1) // _NW + 7) // 8 * 8
    rpad = rpw * _NW
    ei3 = edge_index.astype(jnp.int32).reshape(2, r, 128)
    src_i, dst_i, dsts_i = _tc_prep(ei3, 2 * rpad)
    zeros_tab = jnp.zeros((n, DIM), jnp.float32)

    wx = W_i[:, :DIM].T
    we = W_i[:, DIM:].T
    wh_t = W_h.T
    wox = W_o[:, :DIM].T
    wos = W_o[:, DIM:].T
    b2 = b_o.reshape(1, DIM)

    gathers = [_make_gather(n, eh, h * rpad) for h in (0, 1)]
    scatters = [_make_scatter(n, eh, h * rpad) for h in (0, 1)]

    gx = [gathers[h](x, src_i) for h in (0, 1)]
    inp, msg = zip(*[_tc_init(gx[h], edge_attr, wx, we, h) for h in (0, 1)])
    inp, msg = list(inp), list(msg)
    part = [scatters[h](msg[h], dst_i, zeros_tab) for h in (0, 1)]
    for _ in range(depth - 1):
        esum = _tc_combine(part[0], part[1])
        g0 = gathers[0](esum, dsts_i)
        g1 = gathers[1](esum, dsts_i)
        msg[0] = _tc_depth(msg[0], g0, inp[0], wh_t)
        part[0] = scatters[0](msg[0], dst_i, zeros_tab)
        msg[1] = _tc_depth(msg[1], g1, inp[1], wh_t)
        part[1] = scatters[1](msg[1], dst_i, zeros_tab)
    return _tc_final(x, part[0], part[1], wox, wos, b2)
